# baseline (speedup 1.0000x reference)
import jax
import jax.numpy as jnp
from jax.experimental import pallas as pl
from jax.experimental.pallas import tpu as pltpu


def _make_stage_a(nb1, bm1):
    def _stage_a(x_ref, w1_ref, adj_ref, b1_ref, w2_ref, s2_ref, q_ref, s1_scr):
        i = pl.program_id(0)

        @pl.when(i < nb1)
        def _():
            s1_scr[pl.ds(i * bm1, bm1), :] = jnp.dot(
                x_ref[...].astype(jnp.bfloat16),
                w1_ref[...],
                preferred_element_type=jnp.float32,
            ).astype(jnp.bfloat16)

        @pl.when(i >= nb1)
        def _():
            a = adj_ref[...]
            q_ref[...] = (a * 255.0 + 0.5).astype(jnp.uint8)
            h = jnp.dot(
                a.astype(jnp.bfloat16), s1_scr[...], preferred_element_type=jnp.float32
            )
            h = jnp.maximum(h + b1_ref[...], 0.0)
            s2f = jnp.dot(
                h.astype(jnp.bfloat16), w2_ref[...], preferred_element_type=jnp.float32
            )
            hi = s2f.astype(jnp.bfloat16)
            lo = (s2f - hi.astype(jnp.float32)).astype(jnp.bfloat16)
            s2_ref[...] = jnp.concatenate([hi, lo], axis=1)

    return _stage_a


def _layer2_kernel(q_ref, s2_ref, b2_ref, o_ref):
    zf = jnp.dot(
        q_ref[...].astype(jnp.bfloat16),
        s2_ref[...],
        preferred_element_type=jnp.float32,
    )
    nl = zf.shape[1] // 2
    z = (zf[:, :nl] + zf[:, nl:]) * (1.0 / 255.0)
    o_ref[...] = jax.nn.sigmoid(z + b2_ref[...])


def kernel(x, adj, W1, b1, W2, b2):
    n, nfeat = x.shape
    nhid = W1.shape[1]
    nlabel = W2.shape[1]

    bm1 = 2000 if n % 2000 == 0 else 8
    bm = 400 if n % 400 == 0 else 8
    nb1 = n // bm1
    nb = n // bm

    s2, q = pl.pallas_call(
        _make_stage_a(nb1, bm1),
        grid=(nb1 + nb,),
        in_specs=[
            pl.BlockSpec((bm1, nfeat), lambda i: (jnp.minimum(i, nb1 - 1), 0)),
            pl.BlockSpec((nfeat, nhid), lambda i: (0, 0)),
            pl.BlockSpec((bm, n), lambda i: (jnp.maximum(i - nb1, 0), 0)),
            pl.BlockSpec((1, nhid), lambda i: (0, 0)),
            pl.BlockSpec((nhid, nlabel), lambda i: (0, 0)),
        ],
        out_specs=[
            pl.BlockSpec((bm, 2 * nlabel), lambda i: (jnp.maximum(i - nb1, 0), 0)),
            pl.BlockSpec((bm, n), lambda i: (jnp.maximum(i - nb1, 0), 0)),
        ],
        out_shape=[
            jax.ShapeDtypeStruct((n, 2 * nlabel), jnp.bfloat16),
            jax.ShapeDtypeStruct((n, n), jnp.uint8),
        ],
        scratch_shapes=[pltpu.VMEM((n, nhid), jnp.bfloat16)],
    )(x, W1.astype(jnp.bfloat16), adj, b1.reshape(1, nhid), W2.astype(jnp.bfloat16))

    bm3 = 2000 if n % 2000 == 0 else 8
    out = pl.pallas_call(
        _layer2_kernel,
        grid=(n // bm3,),
        in_specs=[
            pl.BlockSpec((bm3, n), lambda i: (i, 0)),
            pl.BlockSpec((n, 2 * nlabel), lambda i: (0, 0)),
            pl.BlockSpec((1, nlabel), lambda i: (0, 0)),
        ],
        out_specs=pl.BlockSpec((bm3, nlabel), lambda i: (i, 0)),
        out_shape=jax.ShapeDtypeStruct((n, nlabel), jnp.float32),
    )(q, s2, b2.reshape(1, nlabel))
    return out


# speedup vs baseline: 1.0128x; 1.0128x over previous
import jax
import jax.numpy as jnp
from jax.experimental import pallas as pl
from jax.experimental.pallas import tpu as pltpu


def _make_stage_a(nb1, bm1):
    def _stage_a(x_ref, w1_ref, adj_ref, b1_ref, w2_ref, s2_ref, q_ref, s1_scr):
        i = pl.program_id(0)

        @pl.when(i < nb1)
        def _():
            s1_scr[pl.ds(i * bm1, bm1), :] = jnp.dot(
                x_ref[...].astype(jnp.bfloat16),
                w1_ref[...],
                preferred_element_type=jnp.float32,
            ).astype(jnp.bfloat16)

        @pl.when(i >= nb1)
        def _():
            a = adj_ref[...]
            q_ref[...] = (a * 255.0 + 0.5).astype(jnp.uint8)
            h = jnp.dot(
                a.astype(jnp.bfloat16), s1_scr[...], preferred_element_type=jnp.float32
            )
            h = jnp.maximum(h + b1_ref[...], 0.0)
            s2f = jnp.dot(
                h.astype(jnp.bfloat16), w2_ref[...], preferred_element_type=jnp.float32
            )
            hi = s2f.astype(jnp.bfloat16)
            lo = (s2f - hi.astype(jnp.float32)).astype(jnp.bfloat16)
            s2_ref[...] = jnp.concatenate([hi, lo], axis=1)

    return _stage_a


def _layer2_kernel(q_ref, s2_ref, b2_ref, o_ref):
    zf = jnp.dot(
        q_ref[...].astype(jnp.bfloat16),
        s2_ref[...],
        preferred_element_type=jnp.float32,
    )
    nl = zf.shape[1] // 2
    z = (zf[:, :nl] + zf[:, nl:]) * (1.0 / 255.0)
    o_ref[...] = jax.nn.sigmoid(z + b2_ref[...])


def kernel(x, adj, W1, b1, W2, b2):
    n, nfeat = x.shape
    nhid = W1.shape[1]
    nlabel = W2.shape[1]

    bm1 = 2000 if n % 2000 == 0 else 8
    bm = 400 if n % 400 == 0 else 8
    nb1 = n // bm1
    nb = n // bm

    s2, q = pl.pallas_call(
        _make_stage_a(nb1, bm1),
        grid=(nb1 + nb,),
        in_specs=[
            pl.BlockSpec((bm1, nfeat), lambda i: (jnp.minimum(i, nb1 - 1), 0)),
            pl.BlockSpec((nfeat, nhid), lambda i: (0, 0)),
            pl.BlockSpec((bm, n), lambda i: (jnp.maximum(i - nb1, 0), 0)),
            pl.BlockSpec((1, nhid), lambda i: (0, 0)),
            pl.BlockSpec((nhid, nlabel), lambda i: (0, 0)),
        ],
        out_specs=[
            pl.BlockSpec((bm, 2 * nlabel), lambda i: (jnp.maximum(i - nb1, 0), 0)),
            pl.BlockSpec((bm, n), lambda i: (jnp.maximum(i - nb1, 0), 0)),
        ],
        out_shape=[
            jax.ShapeDtypeStruct((n, 2 * nlabel), jnp.bfloat16),
            jax.ShapeDtypeStruct((n, n), jnp.uint8),
        ],
        scratch_shapes=[pltpu.VMEM((n, nhid), jnp.bfloat16)],
    )(x, W1.astype(jnp.bfloat16), adj, b1.reshape(1, nhid), W2.astype(jnp.bfloat16))

    bm3 = 1000 if n % 1000 == 0 else 8
    out = pl.pallas_call(
        _layer2_kernel,
        grid=(n // bm3,),
        in_specs=[
            pl.BlockSpec((bm3, n), lambda i: (i, 0)),
            pl.BlockSpec((n, 2 * nlabel), lambda i: (0, 0)),
            pl.BlockSpec((1, nlabel), lambda i: (0, 0)),
        ],
        out_specs=pl.BlockSpec((bm3, nlabel), lambda i: (i, 0)),
        out_shape=jax.ShapeDtypeStruct((n, nlabel), jnp.float32),
    )(q, s2, b2.reshape(1, nlabel))
    return out
